# Initial kernel scaffold; baseline (speedup 1.0000x reference)
import jax
import jax.numpy as jnp
from jax import lax
from jax.experimental import pallas as pl
from jax.experimental.pallas import tpu as pltpu
from jax.experimental.pallas import tpu_sc as plsc

NUM_ROWS_TBL = 100000
BATCH_N = 16384
LR = 200
LS = 50
NC = 2
NS = 16
NW = NC * NS
ROWS_PER_W = BATCH_N // NW
GROUPS = ROWS_PER_W // 16


def _sc_body(rule_flat, syn_flat, rw_hbm, sw_hbm, gb_hbm, out_hbm,
             table_v, idx_v, sidx_v, acc_v, out_v, gb_v):
    wid = lax.axis_index("s") * NC + lax.axis_index("c")
    base = wid * ROWS_PER_W

    lane = jnp.arange(16, dtype=jnp.int32)
    zero16 = jnp.zeros((16,), jnp.float32)

    pltpu.sync_copy(gb_hbm, gb_v)

    pltpu.sync_copy(rw_hbm.at[pl.ds(0, NUM_ROWS_TBL)], table_v)

    lane_r = lane * LR

    def rule_step(j, acc):
        col = plsc.load_gather(idx_v, [lane_r + j])
        w = plsc.load_gather(table_v, [col])
        return acc + w * w

    for g in range(GROUPS):
        off = pl.multiple_of(base * LR + g * (16 * LR), 8)
        pltpu.sync_copy(rule_flat.at[pl.ds(off, 16 * LR)], idx_v)
        acc = lax.fori_loop(0, LR, rule_step, zero16)
        acc_v[pl.ds(g * 16, 16)] = acc

    pltpu.sync_copy(sw_hbm.at[pl.ds(0, NUM_ROWS_TBL)], table_v)

    lane_s = lane * LS

    def syn_step(j, acc):
        col = plsc.load_gather(sidx_v, [lane_s + j])
        w = plsc.load_gather(table_v, [col])
        return acc + w * w

    gamma = gb_v[pl.ds(0, 16)]
    bias = gb_v[pl.ds(16, 16)]

    for g in range(GROUPS):
        off = pl.multiple_of(base * LS + g * (16 * LS), 8)
        pltpu.sync_copy(syn_flat.at[pl.ds(off, 16 * LS)], sidx_v)
        acc = lax.fori_loop(0, LS, syn_step, acc_v[pl.ds(g * 16, 16)])
        score = gamma * acc + bias
        score = jnp.minimum(jnp.maximum(score, 0.0), 30.0)
        out_v[pl.ds(g * 16, 16)] = 1.0 - jnp.exp(-score)

    pltpu.sync_copy(out_v, out_hbm.at[pl.ds(base, ROWS_PER_W)])


@jax.jit
def _surprisal_sc(rule_flat, syn_flat, rw, sw, gb):
    mesh = plsc.VectorSubcoreMesh(core_axis_name="c", subcore_axis_name="s",
                                  num_cores=NC, num_subcores=NS)
    return pl.kernel(
        _sc_body,
        out_type=jax.ShapeDtypeStruct((BATCH_N,), jnp.float32),
        mesh=mesh,
        scratch_types=[
            pltpu.VMEM((NUM_ROWS_TBL,), jnp.float32),
            pltpu.VMEM((16 * LR,), jnp.int32),
            pltpu.VMEM((16 * LS,), jnp.int32),
            pltpu.VMEM((ROWS_PER_W,), jnp.float32),
            pltpu.VMEM((ROWS_PER_W,), jnp.float32),
            pltpu.VMEM((32,), jnp.float32),
        ],
    )(rule_flat, syn_flat, rw, sw, gb)


def kernel(rule_idx, synergy_idx, rules_w, synergy_w, bias, gamma):
    rule_flat = rule_idx.astype(jnp.int32).reshape(-1)
    syn_flat = synergy_idx.astype(jnp.int32).reshape(-1)
    rw = rules_w.reshape(-1)
    sw = synergy_w.reshape(-1)
    gb = jnp.concatenate([jnp.broadcast_to(gamma, (16,)),
                          jnp.broadcast_to(bias, (16,))])
    return _surprisal_sc(rule_flat, syn_flat, rw, sw, gb)


# speedup vs baseline: 221.9878x; 221.9878x over previous
import jax
import jax.numpy as jnp
from jax import lax
from jax.experimental import pallas as pl
from jax.experimental.pallas import tpu as pltpu
from jax.experimental.pallas import tpu_sc as plsc

NUM_ROWS_TBL = 100000
BATCH_N = 16384
LR = 200
LS = 50
NC = 2
NS = 16
NW = NC * NS
ROWS_PER_W = BATCH_N // NW
GROUPS = ROWS_PER_W // 16


def _sc_body(rule_flat, syn_flat, rw_hbm, sw_hbm, gb_hbm, out_hbm,
             table_v, idx_v, sidx_v, acc_v, out_v, gb_v):
    wid = lax.axis_index("s") * NC + lax.axis_index("c")
    base = wid * ROWS_PER_W

    lane = jnp.arange(16, dtype=jnp.int32)
    zero16 = jnp.zeros((16,), jnp.float32)

    pltpu.sync_copy(gb_hbm, gb_v)

    pltpu.sync_copy(rw_hbm.at[pl.ds(0, NUM_ROWS_TBL)], table_v)

    lane_r = lane * LR

    def rule_step(j, acc):
        col = plsc.load_gather(idx_v, [lane_r + j])
        w = plsc.load_gather(table_v, [col])
        return acc + w * w

    for g in range(GROUPS):
        off = pl.multiple_of(base * LR + g * (16 * LR), 8)
        pltpu.sync_copy(rule_flat.at[pl.ds(off, 16 * LR)], idx_v)
        acc = lax.fori_loop(0, LR, rule_step, zero16)
        acc_v[pl.ds(g * 16, 16)] = acc

    pltpu.sync_copy(sw_hbm.at[pl.ds(0, NUM_ROWS_TBL)], table_v)

    lane_s = lane * LS

    def syn_step(j, acc):
        col = plsc.load_gather(sidx_v, [lane_s + j])
        w = plsc.load_gather(table_v, [col])
        return acc + w * w

    gamma = gb_v[pl.ds(0, 16)]
    bias = gb_v[pl.ds(16, 16)]

    for g in range(GROUPS):
        off = pl.multiple_of(base * LS + g * (16 * LS), 8)
        pltpu.sync_copy(syn_flat.at[pl.ds(off, 16 * LS)], sidx_v)
        acc = lax.fori_loop(0, LS, syn_step, acc_v[pl.ds(g * 16, 16)])
        score = gamma * acc + bias
        score = jnp.minimum(jnp.maximum(score, 0.0), 30.0)
        out_v[pl.ds(g * 16, 16)] = 1.0 - jnp.exp(-score)

    pltpu.sync_copy(out_v, out_hbm.at[pl.ds(base, ROWS_PER_W)])


@jax.jit
def _surprisal_sc(rule_flat, syn_flat, rw, sw, gb):
    mesh = plsc.VectorSubcoreMesh(core_axis_name="c", subcore_axis_name="s",
                                  num_cores=NC, num_subcores=NS)
    return pl.kernel(
        _sc_body,
        out_type=jax.ShapeDtypeStruct((BATCH_N,), jnp.float32),
        mesh=mesh,
        compiler_params=pltpu.CompilerParams(needs_layout_passes=False),
        scratch_types=[
            pltpu.VMEM((NUM_ROWS_TBL,), jnp.float32),
            pltpu.VMEM((16 * LR,), jnp.int32),
            pltpu.VMEM((16 * LS,), jnp.int32),
            pltpu.VMEM((ROWS_PER_W,), jnp.float32),
            pltpu.VMEM((ROWS_PER_W,), jnp.float32),
            pltpu.VMEM((32,), jnp.float32),
        ],
    )(rule_flat, syn_flat, rw, sw, gb)


def kernel(rule_idx, synergy_idx, rules_w, synergy_w, bias, gamma):
    rule_flat = rule_idx.astype(jnp.int32).reshape(-1)
    syn_flat = synergy_idx.astype(jnp.int32).reshape(-1)
    rw = rules_w.reshape(-1)
    sw = synergy_w.reshape(-1)
    gb = jnp.concatenate([jnp.broadcast_to(gamma, (16,)),
                          jnp.broadcast_to(bias, (16,))])
    return _surprisal_sc(rule_flat, syn_flat, rw, sw, gb)


# speedup vs baseline: 279.6657x; 1.2598x over previous
import jax
import jax.numpy as jnp
from jax import lax
from jax.experimental import pallas as pl
from jax.experimental.pallas import tpu as pltpu
from jax.experimental.pallas import tpu_sc as plsc

NUM_ROWS_TBL = 100000
BATCH_N = 16384
LR = 200
LS = 50
NC = 2
NS = 16
NW = NC * NS
ROWS_PER_W = BATCH_N // NW
GROUPS = ROWS_PER_W // 16


def _sc_body(rule_flat, syn_flat, rw_hbm, sw_hbm, gb_hbm, out_hbm,
             table_v, idx_v, sidx_v, acc_v, out_v, gb_v):
    wid = lax.axis_index("s") * NC + lax.axis_index("c")
    base = wid * ROWS_PER_W

    lane = jnp.arange(16, dtype=jnp.int32)
    zero16 = jnp.zeros((16,), jnp.float32)

    pltpu.sync_copy(gb_hbm, gb_v)

    pltpu.sync_copy(rw_hbm.at[pl.ds(0, NUM_ROWS_TBL)], table_v)

    lane_r = lane * LR
    UR = 8

    def rule_step(i, accs):
        j0 = i * UR
        out = []
        for u in range(UR):
            col = plsc.load_gather(idx_v, [lane_r + (j0 + u)])
            w = plsc.load_gather(table_v, [col])
            out.append(accs[u] + w * w)
        return tuple(out)

    for g in range(GROUPS):
        off = pl.multiple_of(base * LR + g * (16 * LR), 8)
        pltpu.sync_copy(rule_flat.at[pl.ds(off, 16 * LR)], idx_v)
        accs = lax.fori_loop(0, LR // UR, rule_step, (zero16,) * UR)
        acc = accs[0]
        for u in range(1, UR):
            acc = acc + accs[u]
        acc_v[pl.ds(g * 16, 16)] = acc

    pltpu.sync_copy(sw_hbm.at[pl.ds(0, NUM_ROWS_TBL)], table_v)

    lane_s = lane * LS
    US = 5

    def syn_step(i, accs):
        j0 = i * US
        out = []
        for u in range(US):
            col = plsc.load_gather(sidx_v, [lane_s + (j0 + u)])
            w = plsc.load_gather(table_v, [col])
            out.append(accs[u] + w * w)
        return tuple(out)

    gamma = gb_v[pl.ds(0, 16)]
    bias = gb_v[pl.ds(16, 16)]

    for g in range(GROUPS):
        off = pl.multiple_of(base * LS + g * (16 * LS), 8)
        pltpu.sync_copy(syn_flat.at[pl.ds(off, 16 * LS)], sidx_v)
        accs = lax.fori_loop(0, LS // US, syn_step,
                             (acc_v[pl.ds(g * 16, 16)],) + (zero16,) * (US - 1))
        acc = accs[0]
        for u in range(1, US):
            acc = acc + accs[u]
        score = gamma * acc + bias
        score = jnp.minimum(jnp.maximum(score, 0.0), 30.0)
        out_v[pl.ds(g * 16, 16)] = 1.0 - jnp.exp(-score)

    pltpu.sync_copy(out_v, out_hbm.at[pl.ds(base, ROWS_PER_W)])


@jax.jit
def _surprisal_sc(rule_flat, syn_flat, rw, sw, gb):
    mesh = plsc.VectorSubcoreMesh(core_axis_name="c", subcore_axis_name="s",
                                  num_cores=NC, num_subcores=NS)
    return pl.kernel(
        _sc_body,
        out_type=jax.ShapeDtypeStruct((BATCH_N,), jnp.float32),
        mesh=mesh,
        compiler_params=pltpu.CompilerParams(needs_layout_passes=False),
        scratch_types=[
            pltpu.VMEM((NUM_ROWS_TBL,), jnp.float32),
            pltpu.VMEM((16 * LR,), jnp.int32),
            pltpu.VMEM((16 * LS,), jnp.int32),
            pltpu.VMEM((ROWS_PER_W,), jnp.float32),
            pltpu.VMEM((ROWS_PER_W,), jnp.float32),
            pltpu.VMEM((32,), jnp.float32),
        ],
    )(rule_flat, syn_flat, rw, sw, gb)


def kernel(rule_idx, synergy_idx, rules_w, synergy_w, bias, gamma):
    rule_flat = rule_idx.astype(jnp.int32).reshape(-1)
    syn_flat = synergy_idx.astype(jnp.int32).reshape(-1)
    rw = rules_w.reshape(-1)
    sw = synergy_w.reshape(-1)
    gb = jnp.concatenate([jnp.broadcast_to(gamma, (16,)),
                          jnp.broadcast_to(bias, (16,))])
    return _surprisal_sc(rule_flat, syn_flat, rw, sw, gb)


# speedup vs baseline: 347.3442x; 1.2420x over previous
import jax
import jax.numpy as jnp
from jax import lax
from jax.experimental import pallas as pl
from jax.experimental.pallas import tpu as pltpu
from jax.experimental.pallas import tpu_sc as plsc

NUM_ROWS_TBL = 100000
BATCH_N = 16384
LR = 200
LS = 50
NC = 2
NS = 16
NW = NC * NS
ROWS_PER_W = BATCH_N // NW
GROUPS = ROWS_PER_W // 16
GPC = 2
CHUNKS = GROUPS // GPC


def _sc_body(rule_flat, syn_flat, rw_hbm, sw_hbm, gb_hbm, out_hbm,
             table_v, ridx_v0, ridx_v1, sidx_v0, sidx_v1, acc_v, gb_v,
             sem0, sem1):
    wid = lax.axis_index("s") * NC + lax.axis_index("c")
    base = wid * ROWS_PER_W

    lane = jnp.arange(16, dtype=jnp.int32)
    zero16 = jnp.zeros((16,), jnp.float32)
    sems = (sem0, sem1)

    pltpu.sync_copy(gb_hbm, gb_v)

    pltpu.sync_copy(rw_hbm.at[pl.ds(0, NUM_ROWS_TBL)], table_v)

    lane_r = lane * LR
    UR = 8
    RCH = GPC * 16 * LR

    def rule_step_for(buf):
        def rule_step(i, accs):
            j0 = i * UR
            out = []
            for u in range(UR):
                col = plsc.load_gather(buf, [lane_r + (j0 + u)])
                w = plsc.load_gather(table_v, [col])
                out.append(accs[u] + w * w)
            return tuple(out)
        return rule_step

    rbufs = (ridx_v0, ridx_v1)

    def rule_dma(c, buf_slot):
        off = pl.multiple_of(base * LR + c * RCH, 8)
        return pltpu.async_copy(rule_flat.at[pl.ds(off, RCH)],
                                rbufs[buf_slot], sems[buf_slot])

    pending = rule_dma(0, 0)
    for c in range(CHUNKS):
        pending.wait()
        if c + 1 < CHUNKS:
            pending = rule_dma(c + 1, (c + 1) % 2)
        for k in range(GPC):
            buf = rbufs[c % 2].at[pl.ds(k * 16 * LR, 16 * LR)]
            accs = lax.fori_loop(0, LR // UR, rule_step_for(buf),
                                 (zero16,) * UR)
            acc = accs[0]
            for u in range(1, UR):
                acc = acc + accs[u]
            acc_v[pl.ds((c * GPC + k) * 16, 16)] = acc

    pltpu.sync_copy(sw_hbm.at[pl.ds(0, NUM_ROWS_TBL)], table_v)

    lane_s = lane * LS
    US = 5
    SCH = GPC * 16 * LS

    def syn_step_for(buf):
        def syn_step(i, accs):
            j0 = i * US
            out = []
            for u in range(US):
                col = plsc.load_gather(buf, [lane_s + (j0 + u)])
                w = plsc.load_gather(table_v, [col])
                out.append(accs[u] + w * w)
            return tuple(out)
        return syn_step

    sbufs = (sidx_v0, sidx_v1)

    def syn_dma(c, buf_slot):
        off = pl.multiple_of(base * LS + c * SCH, 8)
        return pltpu.async_copy(syn_flat.at[pl.ds(off, SCH)],
                                sbufs[buf_slot], sems[buf_slot])

    gamma = gb_v[pl.ds(0, 16)]
    bias = gb_v[pl.ds(16, 16)]

    pending = syn_dma(0, 0)
    for c in range(CHUNKS):
        pending.wait()
        if c + 1 < CHUNKS:
            pending = syn_dma(c + 1, (c + 1) % 2)
        for k in range(GPC):
            g = c * GPC + k
            buf = sbufs[c % 2].at[pl.ds(k * 16 * LS, 16 * LS)]
            accs = lax.fori_loop(0, LS // US, syn_step_for(buf),
                                 (acc_v[pl.ds(g * 16, 16)],) + (zero16,) * (US - 1))
            acc = accs[0]
            for u in range(1, US):
                acc = acc + accs[u]
            score = gamma * acc + bias
            score = jnp.minimum(jnp.maximum(score, 0.0), 30.0)
            acc_v[pl.ds(g * 16, 16)] = 1.0 - jnp.exp(-score)

    pltpu.sync_copy(acc_v, out_hbm.at[pl.ds(base, ROWS_PER_W)])


@jax.jit
def _surprisal_sc(rule_flat, syn_flat, rw, sw, gb):
    mesh = plsc.VectorSubcoreMesh(core_axis_name="c", subcore_axis_name="s",
                                  num_cores=NC, num_subcores=NS)
    return pl.kernel(
        _sc_body,
        out_type=jax.ShapeDtypeStruct((BATCH_N,), jnp.float32),
        mesh=mesh,
        compiler_params=pltpu.CompilerParams(needs_layout_passes=False),
        scratch_types=[
            pltpu.VMEM((NUM_ROWS_TBL,), jnp.float32),
            pltpu.VMEM((GPC * 16 * LR,), jnp.int32),
            pltpu.VMEM((GPC * 16 * LR,), jnp.int32),
            pltpu.VMEM((GPC * 16 * LS,), jnp.int32),
            pltpu.VMEM((GPC * 16 * LS,), jnp.int32),
            pltpu.VMEM((ROWS_PER_W,), jnp.float32),
            pltpu.VMEM((32,), jnp.float32),
            pltpu.SemaphoreType.DMA,
            pltpu.SemaphoreType.DMA,
        ],
    )(rule_flat, syn_flat, rw, sw, gb)


def kernel(rule_idx, synergy_idx, rules_w, synergy_w, bias, gamma):
    rule_flat = rule_idx.astype(jnp.int32).reshape(-1)
    syn_flat = synergy_idx.astype(jnp.int32).reshape(-1)
    rw = rules_w.reshape(-1)
    sw = synergy_w.reshape(-1)
    gb = jnp.concatenate([jnp.broadcast_to(gamma, (16,)),
                          jnp.broadcast_to(bias, (16,))])
    return _surprisal_sc(rule_flat, syn_flat, rw, sw, gb)
